# baseline (speedup 1.0000x reference)
import functools

import jax
import jax.numpy as jnp
from jax.experimental import pallas as pl
from jax.experimental.pallas import tpu as pltpu

B, S, C = 4096, 200, 7


def _loss_kernel(out_ref, lab_ref, mask_ref, rew_ref,
                 loss_ref, corr_ref, nval_ref, accf_ref, acci_ref):
    step = pl.program_id(0)
    nsteps = pl.num_programs(0)

    @pl.when(step == 0)
    def _init():
        accf_ref[0] = 0.0
        accf_ref[1] = 0.0
        acci_ref[0] = 0

    m = mask_ref[...]
    valid = m < 0.5
    validf = valid.astype(jnp.float32)

    bb = m.shape[0]
    loss_part = jnp.float32(0.0)
    correct = valid
    for i in range(3):
        out_i = out_ref[i]
        lab_i = lab_ref[:, :, i]
        rew_i = rew_ref[:, :, i]
        iota = jax.lax.broadcasted_iota(jnp.int32, (bb, S, C), 2)
        sel = iota == lab_i[:, :, None]
        op = jnp.sum(jnp.where(sel, out_i, 0.0), axis=2)
        loss_part += jnp.sum(jnp.log(op) * (rew_i * validf))
        pred = jnp.argmax(out_i, axis=2).astype(jnp.int32)
        correct = jnp.logical_and(correct, pred == lab_i)

    accf_ref[0] += loss_part
    accf_ref[1] += jnp.sum(validf)
    acci_ref[0] += jnp.sum(correct.astype(jnp.int32))

    @pl.when(step == nsteps - 1)
    def _fin():
        nval = accf_ref[1]
        loss_ref[0] = -accf_ref[0] / nval
        corr_ref[0] = acci_ref[0]
        nval_ref[0] = nval.astype(jnp.int32)


@functools.partial(jax.jit, static_argnames=("interpret",))
def _impl(output_list, labels_3, mask, reward, interpret):
    bb = 16
    grid = (B // bb,)
    loss, corr, nval = pl.pallas_call(
        _loss_kernel,
        grid=grid,
        in_specs=[
            pl.BlockSpec((3, bb, S, C), lambda i: (0, i, 0, 0)),
            pl.BlockSpec((bb, S, 3), lambda i: (i, 0, 0)),
            pl.BlockSpec((bb, S), lambda i: (i, 0)),
            pl.BlockSpec((bb, S, 3), lambda i: (i, 0, 0)),
        ],
        out_specs=[
            pl.BlockSpec(memory_space=pltpu.MemorySpace.SMEM),
            pl.BlockSpec(memory_space=pltpu.MemorySpace.SMEM),
            pl.BlockSpec(memory_space=pltpu.MemorySpace.SMEM),
        ],
        out_shape=[
            jax.ShapeDtypeStruct((1,), jnp.float32),
            jax.ShapeDtypeStruct((1,), jnp.int32),
            jax.ShapeDtypeStruct((1,), jnp.int32),
        ],
        scratch_shapes=[
            pltpu.SMEM((2,), jnp.float32),
            pltpu.SMEM((1,), jnp.int32),
        ],
        interpret=interpret,
    )(output_list, labels_3, mask, reward)
    return (loss[0], corr[0], nval[0])


def kernel(output_list, labels_3, mask, reward):
    return _impl(output_list, labels_3, mask, reward, False)


# speedup vs baseline: 20.0938x; 20.0938x over previous
import functools

import jax
import jax.numpy as jnp
from jax.experimental import pallas as pl
from jax.experimental.pallas import tpu as pltpu

B, S, C = 4096, 200, 7


def _loss_kernel(ch_ref, lab_ref, mask_ref, rew_ref,
                 loss_ref, corr_ref, nval_ref, accf_ref, acci_ref):
    step = pl.program_id(0)
    nsteps = pl.num_programs(0)

    @pl.when(step == 0)
    def _init():
        accf_ref[0] = 0.0
        accf_ref[1] = 0.0
        acci_ref[0] = 0

    m = mask_ref[...]
    valid = m < 0.5
    vf = valid.astype(jnp.float32)

    loss_part = jnp.float32(0.0)
    correct = valid
    for i in range(3):
        lab = lab_ref[i]
        rew = rew_ref[i]
        ch0 = ch_ref[i, 0]
        mx = ch0
        g = ch0
        for k in range(1, C):
            chk = ch_ref[i, k]
            mx = jnp.maximum(mx, chk)
            g = jnp.where(lab == k, chk, g)
        loss_part += jnp.sum(jnp.log(g) * (rew * vf))
        correct = jnp.logical_and(correct, g >= mx)

    accf_ref[0] += loss_part
    accf_ref[1] += jnp.sum(vf)
    acci_ref[0] += jnp.sum(correct.astype(jnp.int32))

    @pl.when(step == nsteps - 1)
    def _fin():
        nval = accf_ref[1]
        loss_ref[0] = -accf_ref[0] / nval
        corr_ref[0] = acci_ref[0]
        nval_ref[0] = nval.astype(jnp.int32)


@functools.partial(jax.jit, static_argnames=("interpret",))
def _impl(output_list, labels_3, mask, reward, interpret):
    chans = jnp.transpose(output_list, (0, 3, 1, 2))
    lab_t = jnp.transpose(labels_3, (2, 0, 1))
    rew_t = jnp.transpose(reward, (2, 0, 1))

    bb = 256
    grid = (B // bb,)
    loss, corr, nval = pl.pallas_call(
        _loss_kernel,
        grid=grid,
        in_specs=[
            pl.BlockSpec((3, C, bb, S), lambda i: (0, 0, i, 0)),
            pl.BlockSpec((3, bb, S), lambda i: (0, i, 0)),
            pl.BlockSpec((bb, S), lambda i: (i, 0)),
            pl.BlockSpec((3, bb, S), lambda i: (0, i, 0)),
        ],
        out_specs=[
            pl.BlockSpec(memory_space=pltpu.MemorySpace.SMEM),
            pl.BlockSpec(memory_space=pltpu.MemorySpace.SMEM),
            pl.BlockSpec(memory_space=pltpu.MemorySpace.SMEM),
        ],
        out_shape=[
            jax.ShapeDtypeStruct((1,), jnp.float32),
            jax.ShapeDtypeStruct((1,), jnp.int32),
            jax.ShapeDtypeStruct((1,), jnp.int32),
        ],
        scratch_shapes=[
            pltpu.SMEM((2,), jnp.float32),
            pltpu.SMEM((1,), jnp.int32),
        ],
        interpret=interpret,
    )(chans, lab_t, mask, rew_t)
    return (loss[0], corr[0], nval[0])


def kernel(output_list, labels_3, mask, reward):
    return _impl(output_list, labels_3, mask, reward, False)


# speedup vs baseline: 74.6982x; 3.7175x over previous
import functools

import jax
import jax.numpy as jnp
from jax.experimental import pallas as pl
from jax.experimental.pallas import tpu as pltpu

B, S, C = 4096, 200, 7


def _loss_kernel(ch_ref, lab_ref, mask_ref, rew_ref,
                 loss_ref, corr_ref, nval_ref, accf_ref, acci_ref):
    step = pl.program_id(0)
    nsteps = pl.num_programs(0)

    @pl.when(step == 0)
    def _init():
        accf_ref[0] = 0.0
        accf_ref[1] = 0.0
        acci_ref[0] = 0

    m = mask_ref[...]
    valid = m < 0.5
    vf = valid.astype(jnp.float32)

    loss_part = jnp.float32(0.0)
    correct = valid
    for i in range(3):
        lab = lab_ref[i]
        rew = rew_ref[i]
        ch0 = ch_ref[i, 0]
        mx = ch0
        g = ch0
        for k in range(1, C):
            chk = ch_ref[i, k]
            mx = jnp.maximum(mx, chk)
            g = jnp.where(lab == k, chk, g)
        loss_part += jnp.sum(jnp.log(g) * (rew * vf))
        correct = jnp.logical_and(correct, g >= mx)

    accf_ref[0] += loss_part
    accf_ref[1] += jnp.sum(vf)
    acci_ref[0] += jnp.sum(correct.astype(jnp.int32))

    @pl.when(step == nsteps - 1)
    def _fin():
        nval = accf_ref[1]
        loss_ref[0] = -accf_ref[0] / nval
        corr_ref[0] = acci_ref[0]
        nval_ref[0] = nval.astype(jnp.int32)


@functools.partial(jax.jit, static_argnames=("interpret",))
def _impl(output_list, labels_3, mask, reward, interpret):
    chans = jnp.transpose(output_list, (0, 3, 2, 1))
    lab_t = jnp.transpose(labels_3, (2, 1, 0))
    rew_t = jnp.transpose(reward, (2, 1, 0))
    mask_t = mask.T

    ss = 8
    grid = (S // ss,)
    loss, corr, nval = pl.pallas_call(
        _loss_kernel,
        grid=grid,
        in_specs=[
            pl.BlockSpec((3, C, ss, B), lambda j: (0, 0, j, 0)),
            pl.BlockSpec((3, ss, B), lambda j: (0, j, 0)),
            pl.BlockSpec((ss, B), lambda j: (j, 0)),
            pl.BlockSpec((3, ss, B), lambda j: (0, j, 0)),
        ],
        out_specs=[
            pl.BlockSpec(memory_space=pltpu.MemorySpace.SMEM),
            pl.BlockSpec(memory_space=pltpu.MemorySpace.SMEM),
            pl.BlockSpec(memory_space=pltpu.MemorySpace.SMEM),
        ],
        out_shape=[
            jax.ShapeDtypeStruct((1,), jnp.float32),
            jax.ShapeDtypeStruct((1,), jnp.int32),
            jax.ShapeDtypeStruct((1,), jnp.int32),
        ],
        scratch_shapes=[
            pltpu.SMEM((2,), jnp.float32),
            pltpu.SMEM((1,), jnp.int32),
        ],
        interpret=interpret,
    )(chans, lab_t, mask_t, rew_t)
    return (loss[0], corr[0], nval[0])


def kernel(output_list, labels_3, mask, reward):
    return _impl(output_list, labels_3, mask, reward, False)
